# Initial kernel scaffold; baseline (speedup 1.0000x reference)
import jax
import jax.numpy as jnp
from jax.experimental import pallas as pl


def kernel(item_seq, item_table, pos_table):
    raise NotImplementedError("write your pallas kernel here")


# speedup vs baseline: 1.9171x; 1.9171x over previous
import functools

import jax
import jax.numpy as jnp
from jax import lax
from jax.experimental import pallas as pl
from jax.experimental.pallas import tpu as pltpu
from jax.experimental.pallas import tpu_sc as plsc

ITEM_NUM = 1000000
HIDDEN = 64
MAXLEN = 200
BATCH = 4096

SCALE = float(HIDDEN) ** 0.5

NUM_CORES = 2
NUM_SUBCORES = 16
NW = NUM_CORES * NUM_SUBCORES
TOTAL = BATCH * MAXLEN
PER_W = TOTAL // NW
CHUNK = 128
NCHUNK = PER_W // CHUNK

_mesh = plsc.VectorSubcoreMesh(core_axis_name="c", subcore_axis_name="s")


@functools.partial(
    pl.kernel,
    mesh=_mesh,
    compiler_params=pltpu.CompilerParams(use_tc_tiling_on_sc=False),
    out_type=jax.ShapeDtypeStruct((TOTAL, HIDDEN), jnp.float32),
    scratch_types=[
        pltpu.VMEM((NCHUNK, CHUNK), jnp.int32),
        pltpu.VMEM((MAXLEN, HIDDEN), jnp.float32),
        pltpu.VMEM((CHUNK, HIDDEN), jnp.float32),
        pltpu.SemaphoreType.DMA,
    ],
)
def _embed_kernel(idx_hbm, table_hbm, pos_hbm, out_hbm, idx_v, pos_v, rows_v, sem):
    wid = lax.axis_index("s") * NUM_CORES + lax.axis_index("c")
    base = wid * PER_W
    pltpu.sync_copy(idx_hbm.at[wid], idx_v)
    pltpu.sync_copy(pos_hbm, pos_v)

    def chunk_body(c, carry):
        pltpu.async_copy(table_hbm.at[idx_v.at[c]], rows_v, sem).wait()

        def row_body(j, carry2):
            p = lax.rem(c * CHUNK + j, MAXLEN)
            for h in range(HIDDEN // 16):
                sl = pl.ds(h * 16, 16)
                rows_v[j, sl] = rows_v[j, sl] * SCALE + pos_v[p, sl]
            return carry2

        lax.fori_loop(0, CHUNK, row_body, 0)
        pltpu.sync_copy(rows_v, out_hbm.at[pl.ds(base + c * CHUNK, CHUNK)])
        return carry

    lax.fori_loop(0, NCHUNK, chunk_body, 0)


def kernel(item_seq, item_table, pos_table):
    idx = item_seq.reshape(NW, NCHUNK, CHUNK)
    out = _embed_kernel(idx, item_table, pos_table)
    return out.reshape(BATCH, MAXLEN, HIDDEN)


# speedup vs baseline: 2.2457x; 1.1714x over previous
import functools

import jax
import jax.numpy as jnp
from jax import lax
from jax.experimental import pallas as pl
from jax.experimental.pallas import tpu as pltpu
from jax.experimental.pallas import tpu_sc as plsc

ITEM_NUM = 1000000
HIDDEN = 64
MAXLEN = 200
BATCH = 4096

SCALE = float(HIDDEN) ** 0.5

NUM_CORES = 2
NUM_SUBCORES = 16
NW = NUM_CORES * NUM_SUBCORES
TOTAL = BATCH * MAXLEN
PER_W = TOTAL // NW
CHUNK = 100
NCHUNK = PER_W // CHUNK
NB = 4
LA = 2

_mesh = plsc.VectorSubcoreMesh(core_axis_name="c", subcore_axis_name="s")


@functools.partial(
    pl.kernel,
    mesh=_mesh,
    compiler_params=pltpu.CompilerParams(use_tc_tiling_on_sc=False),
    out_type=jax.ShapeDtypeStruct((TOTAL, HIDDEN), jnp.float32),
    scratch_types=(
        [pltpu.VMEM((NCHUNK, CHUNK), jnp.int32),
         pltpu.VMEM((MAXLEN, HIDDEN), jnp.float32)]
        + [pltpu.VMEM((CHUNK, HIDDEN), jnp.float32) for _ in range(NB)]
        + [pltpu.SemaphoreType.DMA for _ in range(2 * NB)]
    ),
)
def _embed_kernel(idx_hbm, table_hbm, pos_hbm, out_hbm, idx_v, pos_v,
                  b0, b1, b2, b3, g0, g1, g2, g3, s0, s1, s2, s3):
    bufs = [b0, b1, b2, b3]
    gsem = [g0, g1, g2, g3]
    ssem = [s0, s1, s2, s3]
    wid = lax.axis_index("s") * NUM_CORES + lax.axis_index("c")
    base = wid * PER_W
    pltpu.sync_copy(idx_hbm.at[wid], idx_v)
    pltpu.sync_copy(pos_hbm, pos_v)

    def g_start(c, b):
        pltpu.async_copy(table_hbm.at[idx_v.at[c]], bufs[b], gsem[b])

    def g_wait(c, b):
        pltpu.make_async_copy(table_hbm.at[idx_v.at[c]], bufs[b],
                              gsem[b]).wait()

    def out_slice(c):
        return out_hbm.at[pl.ds(base + c * CHUNK, CHUNK)]

    def s_start(c, b):
        pltpu.async_copy(bufs[b], out_slice(c), ssem[b])

    def s_wait(c, b):
        pltpu.make_async_copy(bufs[b], out_slice(c), ssem[b]).wait()

    for c in range(LA):
        g_start(c, c % NB)

    def outer(o, carry):
        for b in range(NB):
            c = o * NB + b
            g_wait(c, b)
            pbase = lax.rem(c, 2) * CHUNK

            def row(j, carry2):
                for h in range(HIDDEN // 16):
                    sl = pl.ds(h * 16, 16)
                    bufs[b][j, sl] = bufs[b][j, sl] * SCALE + pos_v[pbase + j, sl]
                return carry2

            lax.fori_loop(0, CHUNK, row, 0, unroll=2)
            s_start(c, b)
            cg = c + LA
            bg = (b + LA) % NB

            @pl.when(cg < NCHUNK)
            def _():
                @pl.when(cg >= NB)
                def _():
                    s_wait(cg - NB, bg)
                g_start(cg, bg)
        return carry

    lax.fori_loop(0, NCHUNK // NB, outer, 0)
    for k in range(NB):
        c = NCHUNK - NB + k
        s_wait(c, c % NB)


def kernel(item_seq, item_table, pos_table):
    idx = item_seq.reshape(NW, NCHUNK, CHUNK)
    out = _embed_kernel(idx, item_table, pos_table)
    return out.reshape(BATCH, MAXLEN, HIDDEN)


# speedup vs baseline: 2.2516x; 1.0026x over previous
import functools

import jax
import jax.numpy as jnp
from jax import lax
from jax.experimental import pallas as pl
from jax.experimental.pallas import tpu as pltpu
from jax.experimental.pallas import tpu_sc as plsc

ITEM_NUM = 1000000
HIDDEN = 64
MAXLEN = 200
BATCH = 4096

SCALE = float(HIDDEN) ** 0.5

NUM_CORES = 2
NUM_SUBCORES = 16
NW = NUM_CORES * NUM_SUBCORES
SEQ_PER_W = BATCH // NW
HALF = MAXLEN // 2
NB = 4
LA = 2

_mesh = plsc.VectorSubcoreMesh(core_axis_name="c", subcore_axis_name="s")


@functools.partial(
    pl.kernel,
    mesh=_mesh,
    compiler_params=pltpu.CompilerParams(use_tc_tiling_on_sc=False),
    out_type=jax.ShapeDtypeStruct((BATCH, MAXLEN, HIDDEN), jnp.float32),
    scratch_types=(
        [pltpu.VMEM((SEQ_PER_W, MAXLEN), jnp.int32),
         pltpu.VMEM((MAXLEN, HIDDEN), jnp.float32)]
        + [pltpu.VMEM((MAXLEN, HIDDEN), jnp.float32) for _ in range(NB)]
        + [pltpu.SemaphoreType.DMA for _ in range(2 * NB)]
    ),
)
def _embed_kernel(idx_hbm, table_hbm, pos_hbm, out_hbm, idx_v, pos_v,
                  b0, b1, b2, b3, g0, g1, g2, g3, s0, s1, s2, s3):
    bufs = [b0, b1, b2, b3]
    gsem = [g0, g1, g2, g3]
    ssem = [s0, s1, s2, s3]
    wid = lax.axis_index("s") * NUM_CORES + lax.axis_index("c")
    seq_base = wid * SEQ_PER_W
    pltpu.sync_copy(idx_hbm.at[pl.ds(seq_base, SEQ_PER_W)], idx_v)
    pltpu.sync_copy(pos_hbm, pos_v)

    def g_start(c, b):
        pltpu.async_copy(table_hbm.at[idx_v.at[c]], bufs[b], gsem[b])

    def g_wait(c, b):
        pltpu.make_async_copy(table_hbm.at[idx_v.at[c]], bufs[b],
                              gsem[b]).wait()

    def s_start(c, b):
        pltpu.async_copy(bufs[b], out_hbm.at[seq_base + c], ssem[b])

    def s_wait(c, b):
        pltpu.make_async_copy(bufs[b], out_hbm.at[seq_base + c],
                              ssem[b]).wait()

    for c in range(LA):
        g_start(c, c % NB)

    def outer(o, carry):
        for b in range(NB):
            c = o * NB + b
            g_wait(c, b)

            def row(j, carry2):
                for h in range(HIDDEN // 16):
                    sl = pl.ds(h * 16, 16)
                    bufs[b][j, sl] = bufs[b][j, sl] * SCALE + pos_v[j, sl]
                return carry2

            lax.fori_loop(0, MAXLEN, row, 0, unroll=2)
            s_start(c, b)
            cg = c + LA
            bg = (b + LA) % NB

            @pl.when(cg < SEQ_PER_W)
            def _():
                @pl.when(cg >= NB)
                def _():
                    s_wait(cg - NB, bg)
                g_start(cg, bg)
        return carry

    lax.fori_loop(0, SEQ_PER_W // NB, outer, 0)
    for k in range(NB):
        c = SEQ_PER_W - NB + k
        s_wait(c, c % NB)


def kernel(item_seq, item_table, pos_table):
    return _embed_kernel(item_seq, item_table, pos_table)
